# baseline (speedup 1.0000x reference)
import functools

import jax
import jax.numpy as jnp
from jax import lax
from jax.experimental import pallas as pl
from jax.experimental.pallas import tpu as pltpu
from jax.experimental.pallas import tpu_sc as plsc

N = 100000
DIM = 128
ATOM_DIM = 6
MAX_DIS = 10
LANES = 16
NSEG = DIM // LANES

CHUNK = 400
NWORKERS = 32

TCB = 800
SPLITS = (25600, 25600, 24000, 24800)
OFFS = tuple(sum(SPLITS[:i]) for i in range(len(SPLITS)))


def _toff(dvec):
    d_i = dvec[0]
    dc = jnp.where(d_i > 1000, MAX_DIS + 1, jnp.minimum(d_i, MAX_DIS))
    return dc * DIM


def _make_sc_body(nchunks):
    cpw = (nchunks + NWORKERS - 1) // NWORKERS

    def _sc_body(d_hbm, embf_hbm, de_hbm,
                 d_v, t2f_v, out0_v, out1_v, sem0, sem1):
        wid = lax.axis_index("c") * 16 + lax.axis_index("s")

        pltpu.sync_copy(embf_hbm, t2f_v)

        out_bufs = (out0_v, out1_v)
        sems = (sem0, sem1)
        copies = [None, None]

        for t in range(cpw):
            k = jnp.minimum(wid + t * NWORKERS, nchunks - 1)
            base = k * CHUNK
            buf = t % 2
            out_v = out_bufs[buf]

            pltpu.sync_copy(d_hbm.at[pl.ds(base, CHUNK)],
                            d_v.at[pl.ds(0, CHUNK)])
            if copies[buf] is not None:
                copies[buf].wait()

            def row(i, toff, out_v=out_v):
                toff_next = _toff(d_v[pl.ds(i + 1, LANES)])
                segs = [t2f_v[pl.ds(toff + s * LANES, LANES)]
                        for s in range(NSEG)]
                for s in range(NSEG):
                    out_v[i, pl.ds(s * LANES, LANES)] = segs[s]
                return toff_next

            lax.fori_loop(0, CHUNK, row, _toff(d_v[pl.ds(0, LANES)]))
            copies[buf] = pltpu.async_copy(
                out_v, de_hbm.at[pl.ds(base, CHUNK)], sems[buf])

        for c in copies:
            c.wait()

    return _sc_body


def _tc_body(x_ref, de_ref, w_ref, b_ref, out_ref):
    out_ref[...] = (
        jnp.dot(x_ref[...], w_ref[...], preferred_element_type=jnp.float32)
        + de_ref[...] + b_ref[...])


def _tc_body_alias(x_ref, de_ref, w_ref, b_ref, prev_ref, out_ref):
    del prev_ref
    _tc_body(x_ref, de_ref, w_ref, b_ref, out_ref)


@jax.jit
def _run(x, d, W, b2, embf):
    mesh = plsc.VectorSubcoreMesh(core_axis_name="c", subcore_axis_name="s")

    des = []
    for sp, off in zip(SPLITS, OFFS):
        sc_kern = functools.partial(
            pl.kernel,
            mesh=mesh,
            out_type=jax.ShapeDtypeStruct((sp, DIM), jnp.float32),
            scratch_types=[
                pltpu.VMEM((CHUNK + LANES,), jnp.int32),
                pltpu.VMEM(((MAX_DIS + 2) * DIM,), jnp.float32),
                pltpu.VMEM((CHUNK, DIM), jnp.float32),
                pltpu.VMEM((CHUNK, DIM), jnp.float32),
                pltpu.SemaphoreType.DMA,
                pltpu.SemaphoreType.DMA,
            ],
        )(_make_sc_body(sp // CHUNK))
        des.append(sc_kern(lax.dynamic_slice_in_dim(d, off, sp), embf))

    out = None
    for sp, off, de in zip(SPLITS, OFFS, des):
        blk_off = off // TCB
        in_specs = [
            pl.BlockSpec((TCB, ATOM_DIM), lambda i, o=blk_off: (i + o, 0)),
            pl.BlockSpec((TCB, DIM), lambda i: (i, 0)),
            pl.BlockSpec((ATOM_DIM, DIM), lambda i: (0, 0)),
            pl.BlockSpec((1, DIM), lambda i: (0, 0)),
        ]
        args = [x, de, W, b2]
        body = _tc_body
        aliases = {}
        if out is not None:
            in_specs.append(pl.BlockSpec(memory_space=pl.ANY))
            args.append(out)
            body = _tc_body_alias
            aliases = {4: 0}
        out = pl.pallas_call(
            body,
            grid=(sp // TCB,),
            in_specs=in_specs,
            out_specs=pl.BlockSpec((TCB, DIM), lambda i, o=blk_off: (i + o, 0)),
            out_shape=jax.ShapeDtypeStruct((N, DIM), jnp.float32),
            input_output_aliases=aliases,
        )(*args)
    return out


def kernel(x, d, W, b, emb_d):
    return _run(x, d, W, b.reshape(1, DIM), emb_d.reshape(-1))


# speedup vs baseline: 1.2667x; 1.2667x over previous
import functools

import jax
import jax.numpy as jnp
from jax import lax
from jax.experimental import pallas as pl
from jax.experimental.pallas import tpu as pltpu
from jax.experimental.pallas import tpu_sc as plsc

N = 100000
DIM = 128
ATOM_DIM = 6
MAX_DIS = 10
LANES = 16
NSEG = DIM // LANES

CHUNK = 200
NWORKERS = 32

TCB = 5000
SPLITS = (25000, 25000, 25000, 25000)
OFFS = tuple(sum(SPLITS[:i]) for i in range(len(SPLITS)))


def _toff(dvec):
    d_i = dvec[0]
    dc = jnp.where(d_i > 1000, MAX_DIS + 1, jnp.minimum(d_i, MAX_DIS))
    return dc * DIM


def _make_sc_body(nchunks):
    cpw = (nchunks + NWORKERS - 1) // NWORKERS

    def _sc_body(d_hbm, embf_hbm, de_hbm,
                 d_v, t2f_v, out0_v, out1_v, sem0, sem1):
        wid = lax.axis_index("c") * 16 + lax.axis_index("s")

        pltpu.sync_copy(embf_hbm, t2f_v)

        out_bufs = (out0_v, out1_v)
        sems = (sem0, sem1)
        copies = [None, None]

        for t in range(cpw):
            k = jnp.minimum(wid + t * NWORKERS, nchunks - 1)
            base = k * CHUNK
            buf = t % 2
            out_v = out_bufs[buf]

            pltpu.sync_copy(d_hbm.at[pl.ds(base, CHUNK)],
                            d_v.at[pl.ds(0, CHUNK)])
            if copies[buf] is not None:
                copies[buf].wait()

            def row(i, toff, out_v=out_v):
                toff_next = _toff(d_v[pl.ds(i + 1, LANES)])
                segs = [t2f_v[pl.ds(toff + s * LANES, LANES)]
                        for s in range(NSEG)]
                for s in range(NSEG):
                    out_v[i, pl.ds(s * LANES, LANES)] = segs[s]
                return toff_next

            lax.fori_loop(0, CHUNK, row, _toff(d_v[pl.ds(0, LANES)]))
            copies[buf] = pltpu.async_copy(
                out_v, de_hbm.at[pl.ds(base, CHUNK)], sems[buf])

        for c in copies:
            c.wait()

    return _sc_body


def _tc_body(x_ref, de_ref, w_ref, b_ref, out_ref):
    out_ref[...] = (
        jnp.dot(x_ref[...], w_ref[...], preferred_element_type=jnp.float32)
        + de_ref[...] + b_ref[...])


def _tc_body_alias(x_ref, de_ref, w_ref, b_ref, prev_ref, out_ref):
    del prev_ref
    _tc_body(x_ref, de_ref, w_ref, b_ref, out_ref)


@jax.jit
def _run(x, d, W, b2, embf):
    mesh = plsc.VectorSubcoreMesh(core_axis_name="c", subcore_axis_name="s")

    des = []
    for sp, off in zip(SPLITS, OFFS):
        sc_kern = functools.partial(
            pl.kernel,
            mesh=mesh,
            out_type=jax.ShapeDtypeStruct((sp, DIM), jnp.float32),
            scratch_types=[
                pltpu.VMEM((CHUNK + LANES,), jnp.int32),
                pltpu.VMEM(((MAX_DIS + 2) * DIM,), jnp.float32),
                pltpu.VMEM((CHUNK, DIM), jnp.float32),
                pltpu.VMEM((CHUNK, DIM), jnp.float32),
                pltpu.SemaphoreType.DMA,
                pltpu.SemaphoreType.DMA,
            ],
        )(_make_sc_body(sp // CHUNK))
        des.append(sc_kern(lax.dynamic_slice_in_dim(d, off, sp), embf))

    out = None
    for sp, off, de in zip(SPLITS, OFFS, des):
        blk_off = off // TCB
        in_specs = [
            pl.BlockSpec((TCB, ATOM_DIM), lambda i, o=blk_off: (i + o, 0)),
            pl.BlockSpec((TCB, DIM), lambda i: (i, 0)),
            pl.BlockSpec((ATOM_DIM, DIM), lambda i: (0, 0)),
            pl.BlockSpec((1, DIM), lambda i: (0, 0)),
        ]
        args = [x, de, W, b2]
        body = _tc_body
        aliases = {}
        if out is not None:
            in_specs.append(pl.BlockSpec(memory_space=pl.ANY))
            args.append(out)
            body = _tc_body_alias
            aliases = {4: 0}
        out = pl.pallas_call(
            body,
            grid=(sp // TCB,),
            in_specs=in_specs,
            out_specs=pl.BlockSpec((TCB, DIM), lambda i, o=blk_off: (i + o, 0)),
            out_shape=jax.ShapeDtypeStruct((N, DIM), jnp.float32),
            input_output_aliases=aliases,
        )(*args)
    return out


def kernel(x, d, W, b, emb_d):
    return _run(x, d, W, b.reshape(1, DIM), emb_d.reshape(-1))


# speedup vs baseline: 1.4684x; 1.1593x over previous
import functools

import jax
import jax.numpy as jnp
from jax import lax
from jax.experimental import pallas as pl
from jax.experimental.pallas import tpu as pltpu
from jax.experimental.pallas import tpu_sc as plsc

N = 100000
DIM = 128
ATOM_DIM = 6
MAX_DIS = 10
LANES = 16
NSEG = DIM // LANES

CHUNK = 400
NWORKERS = 32

TCB = 5000
SPLITS = (50000, 50000)
OFFS = tuple(sum(SPLITS[:i]) for i in range(len(SPLITS)))


def _toff(dvec):
    d_i = dvec[0]
    dc = jnp.where(d_i > 1000, MAX_DIS + 1, jnp.minimum(d_i, MAX_DIS))
    return dc * DIM


def _make_sc_body(nchunks):
    cpw = (nchunks + NWORKERS - 1) // NWORKERS

    def _sc_body(d_hbm, embf_hbm, de_hbm,
                 d_v, t2f_v, out0_v, out1_v, sem0, sem1):
        wid = lax.axis_index("c") * 16 + lax.axis_index("s")

        pltpu.sync_copy(embf_hbm, t2f_v)

        out_bufs = (out0_v, out1_v)
        sems = (sem0, sem1)
        copies = [None, None]

        for t in range(cpw):
            k = jnp.minimum(wid + t * NWORKERS, nchunks - 1)
            base = k * CHUNK
            buf = t % 2
            out_v = out_bufs[buf]

            pltpu.sync_copy(d_hbm.at[pl.ds(base, CHUNK)],
                            d_v.at[pl.ds(0, CHUNK)])
            if copies[buf] is not None:
                copies[buf].wait()

            def row(i, toff, out_v=out_v):
                toff_next = _toff(d_v[pl.ds(i + 1, LANES)])
                segs = [t2f_v[pl.ds(toff + s * LANES, LANES)]
                        for s in range(NSEG)]
                for s in range(NSEG):
                    out_v[i, pl.ds(s * LANES, LANES)] = segs[s]
                return toff_next

            lax.fori_loop(0, CHUNK, row, _toff(d_v[pl.ds(0, LANES)]))
            copies[buf] = pltpu.async_copy(
                out_v, de_hbm.at[pl.ds(base, CHUNK)], sems[buf])

        for c in copies:
            c.wait()

    return _sc_body


def _tc_body(x_ref, de_ref, w_ref, b_ref, out_ref):
    out_ref[...] = (
        jnp.dot(x_ref[...], w_ref[...], preferred_element_type=jnp.float32)
        + de_ref[...] + b_ref[...])


def _tc_body_alias(x_ref, de_ref, w_ref, b_ref, prev_ref, out_ref):
    del prev_ref
    _tc_body(x_ref, de_ref, w_ref, b_ref, out_ref)


@jax.jit
def _run(x, d, W, b2, embf):
    mesh = plsc.VectorSubcoreMesh(core_axis_name="c", subcore_axis_name="s")

    des = []
    for sp, off in zip(SPLITS, OFFS):
        sc_kern = functools.partial(
            pl.kernel,
            mesh=mesh,
            out_type=jax.ShapeDtypeStruct((sp, DIM), jnp.float32),
            scratch_types=[
                pltpu.VMEM((CHUNK + LANES,), jnp.int32),
                pltpu.VMEM(((MAX_DIS + 2) * DIM,), jnp.float32),
                pltpu.VMEM((CHUNK, DIM), jnp.float32),
                pltpu.VMEM((CHUNK, DIM), jnp.float32),
                pltpu.SemaphoreType.DMA,
                pltpu.SemaphoreType.DMA,
            ],
        )(_make_sc_body(sp // CHUNK))
        des.append(sc_kern(lax.dynamic_slice_in_dim(d, off, sp), embf))

    out = None
    for sp, off, de in zip(SPLITS, OFFS, des):
        blk_off = off // TCB
        in_specs = [
            pl.BlockSpec((TCB, ATOM_DIM), lambda i, o=blk_off: (i + o, 0)),
            pl.BlockSpec((TCB, DIM), lambda i: (i, 0)),
            pl.BlockSpec((ATOM_DIM, DIM), lambda i: (0, 0)),
            pl.BlockSpec((1, DIM), lambda i: (0, 0)),
        ]
        args = [x, de, W, b2]
        body = _tc_body
        aliases = {}
        if out is not None:
            in_specs.append(pl.BlockSpec(memory_space=pl.ANY))
            args.append(out)
            body = _tc_body_alias
            aliases = {4: 0}
        out = pl.pallas_call(
            body,
            grid=(sp // TCB,),
            in_specs=in_specs,
            out_specs=pl.BlockSpec((TCB, DIM), lambda i, o=blk_off: (i + o, 0)),
            out_shape=jax.ShapeDtypeStruct((N, DIM), jnp.float32),
            input_output_aliases=aliases,
        )(*args)
    return out


def kernel(x, d, W, b, emb_d):
    return _run(x, d, W, b.reshape(1, DIM), emb_d.reshape(-1))
